# baseline (speedup 1.0000x reference)
import functools

import jax
import jax.numpy as jnp
from jax.experimental import pallas as pl
from jax.experimental.pallas import tpu as pltpu

HIDDEN = 128
HEADS = 8
HEAD_DIM = HIDDEN // HEADS
N_NODES = 10000
SCALING = HEAD_DIM ** (-0.5)

_ROW_BLK = 1000


def _qkv_body(h_ref, w_ref, b_ref, out_ref):
    out_ref[...] = (
        jnp.dot(h_ref[...], w_ref[...], preferred_element_type=jnp.float32)
        + b_ref[...]
    )


def _qkv_matmul(h, w_cat, b_cat):
    n = h.shape[0]
    grid = n // _ROW_BLK
    return pl.pallas_call(
        _qkv_body,
        grid=(grid,),
        in_specs=[
            pl.BlockSpec((_ROW_BLK, HIDDEN), lambda i: (i, 0)),
            pl.BlockSpec((HIDDEN, 3 * HIDDEN), lambda i: (0, 0)),
            pl.BlockSpec((1, 3 * HIDDEN), lambda i: (0, 0)),
        ],
        out_specs=pl.BlockSpec((_ROW_BLK, 3 * HIDDEN), lambda i: (i, 0)),
        out_shape=jax.ShapeDtypeStruct((n, 3 * HIDDEN), jnp.float32),
    )(h, w_cat, b_cat)


def kernel(h, edge_index, A_val, Wq, bq, Wk, bk, Wv, bv, Wo, bo):
    n = h.shape[0]
    w_cat = jnp.concatenate([Wq.T * SCALING, Wk.T, Wv.T], axis=1)
    b_cat = jnp.concatenate([bq * SCALING, bk, bv])[None, :]
    qkv = _qkv_matmul(h, w_cat, b_cat)
    q = qkv[:, :HIDDEN].reshape(n, HEAD_DIM, HEADS)
    k = qkv[:, HIDDEN:2 * HIDDEN].reshape(n, HEAD_DIM, HEADS)
    v = qkv[:, 2 * HIDDEN:].reshape(n, HEAD_DIM, HEADS)

    row = edge_index[0]
    col = edge_index[1]
    scores = jnp.sum(q[row] * k[col], axis=1) * A_val[:, None]
    m = jax.ops.segment_max(scores, row, num_segments=n)
    m = jnp.where(jnp.isfinite(m), m, 0.0)
    ex = jnp.exp(scores - m[row])
    s = jax.ops.segment_sum(ex, row, num_segments=n)
    attn = ex / jnp.maximum(s[row], 1e-9)
    out = jax.ops.segment_sum(v[col] * attn[:, None, :], row, num_segments=n)
    return out.reshape(n, HIDDEN) @ Wo.T + bo


# speedup vs baseline: 19.5708x; 19.5708x over previous
import functools

import jax
import jax.numpy as jnp
import numpy as np
from jax import lax
from jax.experimental import pallas as pl
from jax.experimental.pallas import tpu as pltpu
from jax.experimental.pallas import tpu_sc as plsc

HIDDEN = 128
HEADS = 8
HEAD_DIM = HIDDEN // HEADS
SCALING = HEAD_DIM ** (-0.5)

NC = 2
NS = 16
NW = NC * NS
LANES = 16
C = 80

_PERM = np.array([d * HEADS + h for h in range(HEADS) for d in range(HEAD_DIM)],
                 dtype=np.int32)


def _dense1_body(h_ref, w_ref, b_ref, sel_ref, a_ref, q_ref, k_ref, v_ref,
                 m_ref):
    h = h_ref[...]
    qkv = jnp.dot(h, w_ref[...], preferred_element_type=jnp.float32) + b_ref[...]
    q = qkv[:, :HIDDEN]
    k = qkv[:, HIDDEN:2 * HIDDEN]
    v = qkv[:, 2 * HIDDEN:]
    q_ref[...] = q
    k_ref[...] = k
    v_ref[...] = v
    sel = sel_ref[...]
    nq2 = jnp.dot(q * q, sel, preferred_element_type=jnp.float32)
    nk2 = jnp.dot(k * k, sel, preferred_element_type=jnp.float32)
    k2max = jnp.max(nk2, axis=0, keepdims=True)
    amax = jnp.max(jnp.abs(a_ref[...]))
    m_ref[...] = jnp.sqrt(nq2 * k2max) * amax


def _dense1(h, w_cat, b_cat, sel, a2d):
    n = h.shape[0]
    f32 = jnp.float32
    return pl.pallas_call(
        _dense1_body,
        out_shape=(
            jax.ShapeDtypeStruct((n, HIDDEN), f32),
            jax.ShapeDtypeStruct((n, HIDDEN), f32),
            jax.ShapeDtypeStruct((n, HIDDEN), f32),
            jax.ShapeDtypeStruct((n, HEADS), f32),
        ),
    )(h, w_cat, b_cat, sel, a2d)


def _dense2_body(p0_ref, p1_ref, w_ref, b_ref, out_ref):
    acc = p0_ref[...] + p1_ref[...]
    out_ref[...] = (
        jnp.dot(acc, w_ref[...], preferred_element_type=jnp.float32) + b_ref[...]
    )


def _dense2(p0, p1, w_eff, b):
    n = p0.shape[0]
    return pl.pallas_call(
        _dense2_body,
        out_shape=jax.ShapeDtypeStruct((n, HIDDEN), jnp.float32),
    )(p0, p1, w_eff, b)


def _sc_a_body(row_hbm, col_hbm, a_hbm, q_hbm, k_hbm, m_hbm,
               ex_hbm, spart_hbm,
               row_v, col_v, a_v, qrows, krows, mrows, exb, zb, s_sh,
               sem0, sem1, sem2):
    n = q_hbm.shape[0]
    n_edges = row_hbm.shape[0]
    e_per_w = n_edges // NW
    nchunks = e_per_w // C

    cid = lax.axis_index("c")
    sid = lax.axis_index("s")
    wid = sid * NC + cid

    iot = lax.iota(jnp.int32, LANES)
    r_off = iot // HEADS
    c_off = iot % HEADS
    zeros = jnp.zeros((LANES,), jnp.float32)

    tiles = n // 8
    tpw = tiles // NS
    extra = tiles - tpw * NS
    t0 = sid * tpw + jnp.minimum(sid, extra)
    myt = tpw + jnp.where(sid < extra, 1, 0)
    r0 = pl.multiple_of(t0 * 8, 8)
    r_tail = pl.multiple_of((t0 + myt) * 8 - 64, 8)

    def zfill(j, carry):
        plsc.store_scatter(zb, [j * 2 + r_off, c_off], zeros)
        return carry
    lax.fori_loop(0, 32, zfill, 0)
    for t in range(9):
        pltpu.sync_copy(zb, s_sh.at[pl.ds(r0 + t * 64, 64)])
    pltpu.sync_copy(zb, s_sh.at[pl.ds(r_tail, 64)])
    plsc.subcore_barrier()

    ebase = wid * e_per_w

    def chunk_body(c, carry):
        base = pl.multiple_of(ebase + c * C, 8)
        pltpu.sync_copy(row_hbm.at[pl.ds(base, C)], row_v)
        pltpu.sync_copy(col_hbm.at[pl.ds(base, C)], col_v)
        pltpu.sync_copy(a_hbm.at[pl.ds(base, C)], a_v)
        cp0 = pltpu.async_copy(q_hbm.at[row_v], qrows, sem0)
        cp1 = pltpu.async_copy(k_hbm.at[col_v], krows, sem1)
        cp2 = pltpu.async_copy(m_hbm.at[row_v], mrows, sem2)
        cp0.wait()
        cp1.wait()
        cp2.wait()

        def blk_body(j, bcarry):
            e_idx = j * LANES + iot
            a_vr = a_v[pl.ds(j * LANES, LANES)]
            for hd in range(HEADS):
                acc = jnp.zeros((LANES,), jnp.float32)
                for d in range(HEAD_DIM):
                    ch = jnp.full((LANES,), hd * HEAD_DIM + d, jnp.int32)
                    qT = plsc.load_gather(qrows, [e_idx, ch])
                    kT = plsc.load_gather(krows, [e_idx, ch])
                    acc = acc + qT * kT
                hvec = jnp.full((LANES,), hd, jnp.int32)
                mvr = plsc.load_gather(mrows, [e_idx, hvec])
                ex = jnp.exp(acc * a_vr - mvr)
                plsc.store_scatter(exb, [e_idx, hvec], ex)
            return bcarry
        lax.fori_loop(0, C // LANES, blk_body, 0)

        pltpu.sync_copy(exb, ex_hbm.at[pl.ds(base, C)])
        pltpu.sync_copy(exb, s_sh.at[row_v], add=True)
        return carry
    lax.fori_loop(0, nchunks, chunk_body, 0)

    plsc.subcore_barrier()
    obase = pl.multiple_of(cid * n + r0, 8)
    otail = pl.multiple_of(cid * n + r_tail, 8)
    for t in range(9):
        pltpu.sync_copy(s_sh.at[pl.ds(r0 + t * 64, 64)],
                        spart_hbm.at[pl.ds(obase + t * 64, 64)])
    pltpu.sync_copy(s_sh.at[pl.ds(r_tail, 64)],
                    spart_hbm.at[pl.ds(otail, 64)])


def _sc_a(row, col, a_val, q, k, m):
    n = q.shape[0]
    n_edges = row.shape[0]
    f32 = jnp.float32
    mesh = plsc.VectorSubcoreMesh(core_axis_name="c", subcore_axis_name="s")
    return pl.kernel(
        _sc_a_body,
        compiler_params=pltpu.CompilerParams(
            use_tc_tiling_on_sc=False, needs_layout_passes=False),
        out_type=(
            jax.ShapeDtypeStruct((n_edges, HEADS), f32),
            jax.ShapeDtypeStruct((2 * n, HEADS), f32),
        ),
        mesh=mesh,
        scratch_types=[
            pltpu.VMEM((C,), jnp.int32),
            pltpu.VMEM((C,), jnp.int32),
            pltpu.VMEM((C,), f32),
            pltpu.VMEM((C, HIDDEN), f32),
            pltpu.VMEM((C, HIDDEN), f32),
            pltpu.VMEM((C, HEADS), f32),
            pltpu.VMEM((C, HEADS), f32),
            pltpu.VMEM((64, HEADS), f32),
            pltpu.VMEM_SHARED((n, HEADS), f32),
            pltpu.SemaphoreType.DMA,
            pltpu.SemaphoreType.DMA,
            pltpu.SemaphoreType.DMA,
        ],
    )(row, col, a_val, q, k, m)


def _sc_b_body(row_hbm, col_hbm, ex_hbm, s0_hbm, s1_hbm, v_hbm,
               opart_hbm,
               row_v, col_v, exb, s0r, s1r, atb, vrows, contrib, zb, out_sh,
               sem0, sem1, sem2):
    n = v_hbm.shape[0]
    n_edges = row_hbm.shape[0]
    e_per_w = n_edges // NW
    nchunks = e_per_w // C

    cid = lax.axis_index("c")
    sid = lax.axis_index("s")
    wid = sid * NC + cid

    iot = lax.iota(jnp.int32, LANES)
    r_off = iot // HEADS
    c_off = iot % HEADS
    zeros = jnp.zeros((LANES,), jnp.float32)

    tiles = n // 8
    tpw = tiles // NS
    extra = tiles - tpw * NS
    t0 = sid * tpw + jnp.minimum(sid, extra)
    myt = tpw + jnp.where(sid < extra, 1, 0)
    r0 = pl.multiple_of(t0 * 8, 8)
    r_tail = pl.multiple_of((t0 + myt) * 8 - 64, 8)

    def zfill(r, carry):
        for t in range(HEADS):
            zb[r, pl.ds(t * LANES, LANES)] = zeros
        return carry
    lax.fori_loop(0, 64, zfill, 0)
    for t in range(9):
        pltpu.sync_copy(zb, out_sh.at[pl.ds(r0 + t * 64, 64)])
    pltpu.sync_copy(zb, out_sh.at[pl.ds(r_tail, 64)])
    plsc.subcore_barrier()

    ebase = wid * e_per_w

    def chunk_body(c, carry):
        base = pl.multiple_of(ebase + c * C, 8)
        pltpu.sync_copy(row_hbm.at[pl.ds(base, C)], row_v)
        pltpu.sync_copy(col_hbm.at[pl.ds(base, C)], col_v)
        cp0 = pltpu.async_copy(v_hbm.at[col_v], vrows, sem0)
        cp1 = pltpu.async_copy(s0_hbm.at[row_v], s0r, sem1)
        cp2 = pltpu.async_copy(s1_hbm.at[row_v], s1r, sem2)
        pltpu.sync_copy(ex_hbm.at[pl.ds(base, C)], exb)
        cp0.wait()
        cp1.wait()
        cp2.wait()

        def vbody(j, vcarry):
            idx_r = j * 2 + r_off
            ex = plsc.load_gather(exb, [idx_r, c_off])
            s0 = plsc.load_gather(s0r, [idx_r, c_off])
            s1 = plsc.load_gather(s1r, [idx_r, c_off])
            at = ex / jnp.maximum(s0 + s1, 1e-9)
            atb[pl.ds(j * LANES, LANES)] = at
            return vcarry
        lax.fori_loop(0, C * HEADS // LANES, vbody, 0)

        def pair_body(j, pcarry):
            at16 = atb[pl.ds(j * LANES, LANES)]
            for par in range(2):
                e = j * 2 + par
                for hd in range(HEADS):
                    a = at16[par * HEADS + hd]
                    contrib[e, pl.ds(hd * HEAD_DIM, HEAD_DIM)] = (
                        vrows[e, pl.ds(hd * HEAD_DIM, HEAD_DIM)] * a)
            return pcarry
        lax.fori_loop(0, C // 2, pair_body, 0)

        pltpu.sync_copy(contrib, out_sh.at[row_v], add=True)
        return carry
    lax.fori_loop(0, nchunks, chunk_body, 0)

    plsc.subcore_barrier()
    obase = pl.multiple_of(cid * n + r0, 8)
    otail = pl.multiple_of(cid * n + r_tail, 8)
    for t in range(9):
        pltpu.sync_copy(out_sh.at[pl.ds(r0 + t * 64, 64)],
                        opart_hbm.at[pl.ds(obase + t * 64, 64)])
    pltpu.sync_copy(out_sh.at[pl.ds(r_tail, 64)],
                    opart_hbm.at[pl.ds(otail, 64)])


def _sc_b(row, col, ex, s0, s1, v):
    n = v.shape[0]
    f32 = jnp.float32
    mesh = plsc.VectorSubcoreMesh(core_axis_name="c", subcore_axis_name="s")
    return pl.kernel(
        _sc_b_body,
        compiler_params=pltpu.CompilerParams(
            use_tc_tiling_on_sc=False, needs_layout_passes=False),
        out_type=jax.ShapeDtypeStruct((2 * n, HIDDEN), f32),
        mesh=mesh,
        scratch_types=[
            pltpu.VMEM((C,), jnp.int32),
            pltpu.VMEM((C,), jnp.int32),
            pltpu.VMEM((C, HEADS), f32),
            pltpu.VMEM((C, HEADS), f32),
            pltpu.VMEM((C, HEADS), f32),
            pltpu.VMEM((C * HEADS,), f32),
            pltpu.VMEM((C, HIDDEN), f32),
            pltpu.VMEM((C, HIDDEN), f32),
            pltpu.VMEM((64, HIDDEN), f32),
            pltpu.VMEM_SHARED((n, HIDDEN), f32),
            pltpu.SemaphoreType.DMA,
            pltpu.SemaphoreType.DMA,
            pltpu.SemaphoreType.DMA,
        ],
    )(row, col, ex, s0, s1, v)


def kernel(h, edge_index, A_val, Wq, bq, Wk, bk, Wv, bv, Wo, bo):
    n = h.shape[0]
    perm = jnp.asarray(_PERM)
    wq = (Wq * SCALING)[perm, :]
    bqp = (bq * SCALING)[perm]
    wk = Wk[perm, :]
    bkp = bk[perm]
    wv = Wv[perm, :]
    bvp = bv[perm]
    w_cat = jnp.concatenate([wq.T, wk.T, wv.T], axis=1)
    b_cat = jnp.concatenate([bqp, bkp, bvp])[None, :]
    sel = jnp.asarray(
        (np.arange(HIDDEN)[:, None] // HEAD_DIM == np.arange(HEADS)[None, :])
        .astype(np.float32))
    a2d = A_val.reshape(-1, HIDDEN)

    q, k, v, m = _dense1(h, w_cat, b_cat, sel, a2d)

    row = edge_index[0]
    col = edge_index[1]
    ex, spart = _sc_a(row, col, A_val, q, k, m)
    opart = _sc_b(row, col, ex, spart[:n], spart[n:], v)

    w_eff = Wo.T[perm, :]
    return _dense2(opart[:n], opart[n:], w_eff, bo[None, :])


# speedup vs baseline: 24.0300x; 1.2278x over previous
import functools

import jax
import jax.numpy as jnp
import numpy as np
from jax import lax
from jax.experimental import pallas as pl
from jax.experimental.pallas import tpu as pltpu
from jax.experimental.pallas import tpu_sc as plsc

HIDDEN = 128
HEADS = 8
HEAD_DIM = HIDDEN // HEADS
SCALING = HEAD_DIM ** (-0.5)

NC = 2
NS = 16
NW = NC * NS
LANES = 16
C = 80
NCHUNK = 125

_PERM = np.array([d * HEADS + h for h in range(HEADS) for d in range(HEAD_DIM)],
                 dtype=np.int32)

_SC_PARAMS = pltpu.CompilerParams(
    use_tc_tiling_on_sc=False, needs_layout_passes=False)


def _dense1_body(h_ref, w_ref, b_ref, sel_ref, a_ref, q_ref, k_ref, v_ref,
                 m_ref):
    h = h_ref[...]
    qkv = jnp.dot(h, w_ref[...], preferred_element_type=jnp.float32) + b_ref[...]
    q = qkv[:, :HIDDEN]
    k = qkv[:, HIDDEN:2 * HIDDEN]
    v = qkv[:, 2 * HIDDEN:]
    q_ref[...] = q
    k_ref[...] = k
    v_ref[...] = v
    sel = sel_ref[...]
    nq2 = jnp.dot(q * q, sel, preferred_element_type=jnp.float32)
    nk2 = jnp.dot(k * k, sel, preferred_element_type=jnp.float32)
    k2max = jnp.max(nk2, axis=0, keepdims=True)
    amax = jnp.max(jnp.abs(a_ref[...]))
    m_ref[...] = jnp.sqrt(nq2 * k2max) * amax


def _dense1(h, w_cat, b_cat, sel, a2d):
    n = h.shape[0]
    f32 = jnp.float32
    return pl.pallas_call(
        _dense1_body,
        out_shape=(
            jax.ShapeDtypeStruct((n, HIDDEN), f32),
            jax.ShapeDtypeStruct((n, HIDDEN), f32),
            jax.ShapeDtypeStruct((n, HIDDEN), f32),
            jax.ShapeDtypeStruct((n, HEADS), f32),
        ),
    )(h, w_cat, b_cat, sel, a2d)


def _dense2_body(p0_ref, p1_ref, w_ref, b_ref, out_ref):
    acc = p0_ref[...] + p1_ref[...]
    out_ref[...] = (
        jnp.dot(acc, w_ref[...], preferred_element_type=jnp.float32) + b_ref[...]
    )


def _dense2(p0, p1, w_eff, b):
    n = p0.shape[0]
    return pl.pallas_call(
        _dense2_body,
        out_shape=jax.ShapeDtypeStruct((n, HIDDEN), jnp.float32),
    )(p0, p1, w_eff, b)


def _sc_a_body(row_hbm, col_hbm, a_hbm, q_hbm, k_hbm, m_hbm,
               ex_hbm, spart_hbm,
               rows_all, cols_all, a_all,
               qrows0, qrows1, krows0, krows1, mrows0, mrows1, exb0, exb1,
               zb, s_sh,
               semi0, semi1):
    n = q_hbm.shape[0]
    qrows = (qrows0, qrows1)
    krows = (krows0, krows1)
    mrows = (mrows0, mrows1)
    exb = (exb0, exb1)
    semi = (semi0, semi1)

    cid = lax.axis_index("c")
    sid = lax.axis_index("s")
    wid = sid * NC + cid

    iot = lax.iota(jnp.int32, LANES)
    r_off = iot // HEADS
    c_off = iot % HEADS
    zeros = jnp.zeros((LANES,), jnp.float32)

    tiles = n // 8
    tpw = tiles // NS
    extra = tiles - tpw * NS
    t0 = sid * tpw + jnp.minimum(sid, extra)
    myt = tpw + jnp.where(sid < extra, 1, 0)
    r0 = pl.multiple_of(t0 * 8, 8)
    r_tail = pl.multiple_of((t0 + myt) * 8 - 64, 8)

    def zfill(j, carry):
        plsc.store_scatter(zb, [j * 2 + r_off, c_off], zeros)
        return carry
    lax.fori_loop(0, 32, zfill, 0)
    for t in range(9):
        pltpu.sync_copy(zb, s_sh.at[pl.ds(r0 + t * 64, 64)])
    pltpu.sync_copy(zb, s_sh.at[pl.ds(r_tail, 64)])

    cb = wid * NCHUNK
    pltpu.sync_copy(row_hbm.at[pl.ds(cb, NCHUNK)], rows_all)
    pltpu.sync_copy(col_hbm.at[pl.ds(cb, NCHUNK)], cols_all)
    pltpu.sync_copy(a_hbm.at[pl.ds(cb, NCHUNK)], a_all)
    plsc.subcore_barrier()

    ebase = wid * (NCHUNK * C)

    def issue_in(b, c):
        pltpu.async_copy(q_hbm.at[rows_all.at[c]], qrows[b], semi[b])
        pltpu.async_copy(k_hbm.at[cols_all.at[c]], krows[b], semi[b])
        pltpu.async_copy(m_hbm.at[rows_all.at[c]], mrows[b], semi[b])

    def wait_in(b, c):
        pltpu.make_async_copy(q_hbm.at[rows_all.at[c]], qrows[b], semi[b]).wait()
        pltpu.make_async_copy(k_hbm.at[cols_all.at[c]], krows[b], semi[b]).wait()
        pltpu.make_async_copy(m_hbm.at[rows_all.at[c]], mrows[b], semi[b]).wait()

    def do_out(b, c):
        base = pl.multiple_of(ebase + c * C, 8)
        pltpu.sync_copy(exb[b], ex_hbm.at[pl.ds(base, C)])
        pltpu.sync_copy(exb[b], s_sh.at[rows_all.at[c]], add=True)

    def compute(b, c):
        def blk_body(j, bcarry):
            e_idx = j * LANES + iot
            a_vr = a_all[c, pl.ds(j * LANES, LANES)]
            for hd in range(HEADS):
                acc = jnp.zeros((LANES,), jnp.float32)
                for d in range(HEAD_DIM):
                    ch = jnp.full((LANES,), hd * HEAD_DIM + d, jnp.int32)
                    qT = plsc.load_gather(qrows[b], [e_idx, ch])
                    kT = plsc.load_gather(krows[b], [e_idx, ch])
                    acc = acc + qT * kT
                hvec = jnp.full((LANES,), hd, jnp.int32)
                mvr = plsc.load_gather(mrows[b], [e_idx, hvec])
                ex = jnp.exp(acc * a_vr - mvr)
                plsc.store_scatter(exb[b], [e_idx, hvec], ex)
            return bcarry
        lax.fori_loop(0, C // LANES, blk_body, 0)

    issue_in(0, 0)
    issue_in(1, 1)

    def g_body(g, carry):
        for b in range(2):
            c = g * 2 + b
            wait_in(b, c)
            compute(b, c)
            do_out(b, c)
            if b == 0:
                issue_in(b, c + 2)
            else:
                @pl.when(g <= (NCHUNK - 1) // 2 - 2)
                def _():
                    issue_in(b, c + 2)
        return carry
    lax.fori_loop(0, (NCHUNK - 1) // 2, g_body, 0)

    wait_in(0, NCHUNK - 1)
    compute(0, NCHUNK - 1)
    do_out(0, NCHUNK - 1)

    plsc.subcore_barrier()
    obase = pl.multiple_of(cid * n + r0, 8)
    otail = pl.multiple_of(cid * n + r_tail, 8)
    for t in range(9):
        pltpu.sync_copy(s_sh.at[pl.ds(r0 + t * 64, 64)],
                        spart_hbm.at[pl.ds(obase + t * 64, 64)])
    pltpu.sync_copy(s_sh.at[pl.ds(r_tail, 64)],
                    spart_hbm.at[pl.ds(otail, 64)])


def _sc_a(row2, col2, a2, q, k, m):
    n = q.shape[0]
    n_edges = row2.shape[0] * row2.shape[1]
    f32 = jnp.float32
    i32 = jnp.int32
    mesh = plsc.VectorSubcoreMesh(core_axis_name="c", subcore_axis_name="s")
    return pl.kernel(
        _sc_a_body,
        compiler_params=_SC_PARAMS,
        out_type=(
            jax.ShapeDtypeStruct((n_edges, HEADS), f32),
            jax.ShapeDtypeStruct((2 * n, HEADS), f32),
        ),
        mesh=mesh,
        scratch_types=[
            pltpu.VMEM((NCHUNK, C), i32),
            pltpu.VMEM((NCHUNK, C), i32),
            pltpu.VMEM((NCHUNK, C), f32),
            pltpu.VMEM((C, HIDDEN), f32),
            pltpu.VMEM((C, HIDDEN), f32),
            pltpu.VMEM((C, HIDDEN), f32),
            pltpu.VMEM((C, HIDDEN), f32),
            pltpu.VMEM((C, HEADS), f32),
            pltpu.VMEM((C, HEADS), f32),
            pltpu.VMEM((C, HEADS), f32),
            pltpu.VMEM((C, HEADS), f32),
            pltpu.VMEM((64, HEADS), f32),
            pltpu.VMEM_SHARED((n, HEADS), f32),
            pltpu.SemaphoreType.DMA,
            pltpu.SemaphoreType.DMA,
        ],
    )(row2, col2, a2, q, k, m)


def _sc_b_body(row_hbm, col_hbm, ex_hbm, s0_hbm, s1_hbm, v_hbm,
               opart_hbm,
               rb0, rb1, rb2, rb3, cb0, cb1, cb2, cb3,
               vrows0, vrows1, s0r0, s0r1, s1r0, s1r1, exb0, exb1,
               con0, con1, atb, zb, out_sh,
               semx0, semx1, semx2, semx3, semi0, semi1):
    n = v_hbm.shape[0]
    rowb = (rb0, rb1, rb2, rb3)
    colb = (cb0, cb1, cb2, cb3)
    vrows = (vrows0, vrows1)
    s0r = (s0r0, s0r1)
    s1r = (s1r0, s1r1)
    exb = (exb0, exb1)
    con = (con0, con1)
    semx = (semx0, semx1, semx2, semx3)
    semi = (semi0, semi1)

    cid = lax.axis_index("c")
    sid = lax.axis_index("s")
    wid = sid * NC + cid

    iot = lax.iota(jnp.int32, LANES)
    r_off = iot // HEADS
    c_off = iot % HEADS
    zeros = jnp.zeros((LANES,), jnp.float32)

    tiles = n // 8
    tpw = tiles // NS
    extra = tiles - tpw * NS
    t0 = sid * tpw + jnp.minimum(sid, extra)
    myt = tpw + jnp.where(sid < extra, 1, 0)
    r0 = pl.multiple_of(t0 * 8, 8)
    r_tail = pl.multiple_of((t0 + myt) * 8 - 32, 8)

    def zfill(r, carry):
        for t in range(HEADS):
            zb[r, pl.ds(t * LANES, LANES)] = zeros
        return carry
    lax.fori_loop(0, 32, zfill, 0)
    for t in range(19):
        pltpu.sync_copy(zb, out_sh.at[pl.ds(r0 + t * 32, 32)])
    pltpu.sync_copy(zb, out_sh.at[pl.ds(r_tail, 32)])
    plsc.subcore_barrier()

    cbase = wid * NCHUNK
    ebase = wid * (NCHUNK * C)

    def issue_idx(s, c):
        pltpu.async_copy(row_hbm.at[cbase + c], rowb[s], semx[s])
        pltpu.async_copy(col_hbm.at[cbase + c], colb[s], semx[s])

    def wait_idx(s, c):
        pltpu.make_async_copy(row_hbm.at[cbase + c], rowb[s], semx[s]).wait()
        pltpu.make_async_copy(col_hbm.at[cbase + c], colb[s], semx[s]).wait()

    def issue_in(b, s, c):
        base = pl.multiple_of(ebase + c * C, 8)
        pltpu.async_copy(v_hbm.at[colb[s]], vrows[b], semi[b])
        pltpu.async_copy(s0_hbm.at[rowb[s]], s0r[b], semi[b])
        pltpu.async_copy(s1_hbm.at[rowb[s]], s1r[b], semi[b])
        pltpu.async_copy(ex_hbm.at[pl.ds(base, C)], exb[b], semi[b])

    def wait_in(b, s, c):
        base = pl.multiple_of(ebase + c * C, 8)
        pltpu.make_async_copy(v_hbm.at[colb[s]], vrows[b], semi[b]).wait()
        pltpu.make_async_copy(s0_hbm.at[rowb[s]], s0r[b], semi[b]).wait()
        pltpu.make_async_copy(s1_hbm.at[rowb[s]], s1r[b], semi[b]).wait()
        pltpu.make_async_copy(ex_hbm.at[pl.ds(base, C)], exb[b], semi[b]).wait()

    def do_out(b, s):
        pltpu.sync_copy(con[b], out_sh.at[rowb[s]], add=True)

    def compute(b):
        def vbody(j, vcarry):
            idx_r = j * 2 + r_off
            ex = plsc.load_gather(exb[b], [idx_r, c_off])
            s0 = plsc.load_gather(s0r[b], [idx_r, c_off])
            s1 = plsc.load_gather(s1r[b], [idx_r, c_off])
            at = ex / jnp.maximum(s0 + s1, 1e-9)
            atb[pl.ds(j * LANES, LANES)] = at
            return vcarry
        lax.fori_loop(0, C * HEADS // LANES, vbody, 0)

        def pair_body(j, pcarry):
            at16 = atb[pl.ds(j * LANES, LANES)]
            for par in range(2):
                e = j * 2 + par
                for hd in range(HEADS):
                    a = at16[par * HEADS + hd]
                    con[b][e, pl.ds(hd * HEAD_DIM, HEAD_DIM)] = (
                        vrows[b][e, pl.ds(hd * HEAD_DIM, HEAD_DIM)] * a)
            return pcarry
        lax.fori_loop(0, C // 2, pair_body, 0)

    issue_idx(0, 0)
    issue_idx(1, 1)
    wait_idx(0, 0)
    issue_in(0, 0, 0)

    def g_body(g, carry):
        for u in range(4):
            c = g * 4 + u
            b = u % 2
            @pl.when(c <= NCHUNK - 3)
            def _():
                issue_idx((u + 2) % 4, c + 2)
            wait_in(b, u, c)
            compute(b)
            do_out(b, u)
            wait_idx((u + 1) % 4, c + 1)
            issue_in(1 - b, (u + 1) % 4, c + 1)
        return carry
    lax.fori_loop(0, (NCHUNK - 1) // 4, g_body, 0)

    wait_in(0, 0, NCHUNK - 1)
    compute(0)
    do_out(0, 0)

    plsc.subcore_barrier()
    obase = pl.multiple_of(cid * n + r0, 8)
    otail = pl.multiple_of(cid * n + r_tail, 8)
    for t in range(19):
        pltpu.sync_copy(out_sh.at[pl.ds(r0 + t * 32, 32)],
                        opart_hbm.at[pl.ds(obase + t * 32, 32)])
    pltpu.sync_copy(out_sh.at[pl.ds(r_tail, 32)],
                    opart_hbm.at[pl.ds(otail, 32)])


def _sc_b(row2, col2, ex, s0, s1, v):
    n = v.shape[0]
    f32 = jnp.float32
    i32 = jnp.int32
    mesh = plsc.VectorSubcoreMesh(core_axis_name="c", subcore_axis_name="s")
    return pl.kernel(
        _sc_b_body,
        compiler_params=_SC_PARAMS,
        out_type=jax.ShapeDtypeStruct((2 * n, HIDDEN), f32),
        mesh=mesh,
        scratch_types=(
            [pltpu.VMEM((C,), i32) for _ in range(8)]
            + [
                pltpu.VMEM((C, HIDDEN), f32),
                pltpu.VMEM((C, HIDDEN), f32),
                pltpu.VMEM((C, HEADS), f32),
                pltpu.VMEM((C, HEADS), f32),
                pltpu.VMEM((C, HEADS), f32),
                pltpu.VMEM((C, HEADS), f32),
                pltpu.VMEM((C, HEADS), f32),
                pltpu.VMEM((C, HEADS), f32),
                pltpu.VMEM((C, HIDDEN), f32),
                pltpu.VMEM((C, HIDDEN), f32),
                pltpu.VMEM((C * HEADS,), f32),
                pltpu.VMEM((32, HIDDEN), f32),
                pltpu.VMEM_SHARED((n, HIDDEN), f32),
            ]
            + [pltpu.SemaphoreType.DMA for _ in range(6)]
        ),
    )(row2, col2, ex, s0, s1, v)


def kernel(h, edge_index, A_val, Wq, bq, Wk, bk, Wv, bv, Wo, bo):
    n = h.shape[0]
    perm = jnp.asarray(_PERM)
    wq = (Wq * SCALING)[perm, :]
    bqp = (bq * SCALING)[perm]
    wk = Wk[perm, :]
    bkp = bk[perm]
    wv = Wv[perm, :]
    bvp = bv[perm]
    w_cat = jnp.concatenate([wq.T, wk.T, wv.T], axis=1)
    b_cat = jnp.concatenate([bqp, bkp, bvp])[None, :]
    sel = jnp.asarray(
        (np.arange(HIDDEN)[:, None] // HEAD_DIM == np.arange(HEADS)[None, :])
        .astype(np.float32))
    a2d = A_val.reshape(-1, HIDDEN)

    q, k, v, m = _dense1(h, w_cat, b_cat, sel, a2d)

    row2 = edge_index[0].reshape(NW * NCHUNK, C)
    col2 = edge_index[1].reshape(NW * NCHUNK, C)
    a2 = A_val.reshape(NW * NCHUNK, C)
    ex, spart = _sc_a(row2, col2, a2, q, k, m)
    opart = _sc_b(row2, col2, ex, spart[:n], spart[n:], v)

    w_eff = Wo.T[perm, :]
    return _dense2(opart[:n], opart[n:], w_eff, bo[None, :])


# speedup vs baseline: 37.8716x; 1.5760x over previous
import functools

import jax
import jax.numpy as jnp
import numpy as np
from jax import lax
from jax.experimental import pallas as pl
from jax.experimental.pallas import tpu as pltpu
from jax.experimental.pallas import tpu_sc as plsc

HIDDEN = 128
HEADS = 8
HEAD_DIM = HIDDEN // HEADS
SCALING = HEAD_DIM ** (-0.5)

NC = 2
NS = 16
NW = NC * NS
LANES = 16
C = 80
NCHUNK = 125

_PERM = np.array([d * HEADS + h for h in range(HEADS) for d in range(HEAD_DIM)],
                 dtype=np.int32)

_SC_PARAMS = pltpu.CompilerParams(
    use_tc_tiling_on_sc=False, needs_layout_passes=False)


def _dense1_body(h_ref, w_ref, b_ref, sel_ref, a_ref, q_ref, k_ref, v_ref,
                 m_ref):
    h = h_ref[...]
    qkv = jnp.dot(h, w_ref[...], preferred_element_type=jnp.float32) + b_ref[...]
    q = qkv[:, :HIDDEN]
    k = qkv[:, HIDDEN:2 * HIDDEN]
    v = qkv[:, 2 * HIDDEN:]
    n = h.shape[0]
    pad1 = jnp.zeros((n, 1), jnp.float32)
    q_ref[...] = jnp.concatenate([q, pad1], axis=1)
    k_ref[...] = jnp.concatenate([k, pad1], axis=1)
    v_ref[...] = v
    sel = sel_ref[...]
    nq2 = jnp.dot(q * q, sel, preferred_element_type=jnp.float32)
    nk2 = jnp.dot(k * k, sel, preferred_element_type=jnp.float32)
    k2max = jnp.max(nk2, axis=0, keepdims=True)
    amax = jnp.max(jnp.abs(a_ref[...]))
    m_ref[...] = jnp.concatenate(
        [jnp.sqrt(nq2 * k2max) * amax, pad1], axis=1)


def _dense1(h, w_cat, b_cat, sel, a2d):
    n = h.shape[0]
    f32 = jnp.float32
    return pl.pallas_call(
        _dense1_body,
        out_shape=(
            jax.ShapeDtypeStruct((n, HIDDEN + 1), f32),
            jax.ShapeDtypeStruct((n, HIDDEN + 1), f32),
            jax.ShapeDtypeStruct((n, HIDDEN), f32),
            jax.ShapeDtypeStruct((n, HEADS + 1), f32),
        ),
    )(h, w_cat, b_cat, sel, a2d)


def _dense2_body(p0_ref, p1_ref, w_ref, b_ref, out_ref):
    acc = p0_ref[...] + p1_ref[...]
    out_ref[...] = (
        jnp.dot(acc, w_ref[...], preferred_element_type=jnp.float32) + b_ref[...]
    )


def _dense2(p0, p1, w_eff, b):
    n = p0.shape[0]
    return pl.pallas_call(
        _dense2_body,
        out_shape=jax.ShapeDtypeStruct((n, HIDDEN), jnp.float32),
    )(p0, p1, w_eff, b)


def _sc_a_body(row_hbm, col_hbm, a_hbm, q_hbm, k_hbm, m_hbm,
               ex_hbm, spart_hbm,
               rows_all, cols_all, a_all,
               qrows0, qrows1, krows0, krows1, mrows0, mrows1, exb0, exb1,
               zb, s_sh,
               semi0, semi1):
    n = q_hbm.shape[0]
    qrows = (qrows0, qrows1)
    krows = (krows0, krows1)
    mrows = (mrows0, mrows1)
    exb = (exb0, exb1)
    semi = (semi0, semi1)

    cid = lax.axis_index("c")
    sid = lax.axis_index("s")
    wid = sid * NC + cid

    iot = lax.iota(jnp.int32, LANES)
    r_off = iot // HEADS
    c_off = iot % HEADS
    zeros = jnp.zeros((LANES,), jnp.float32)

    tiles = n // 8
    tpw = tiles // NS
    extra = tiles - tpw * NS
    t0 = sid * tpw + jnp.minimum(sid, extra)
    myt = tpw + jnp.where(sid < extra, 1, 0)
    r0 = pl.multiple_of(t0 * 8, 8)
    r_tail = pl.multiple_of((t0 + myt) * 8 - 64, 8)

    def zfill(j, carry):
        plsc.store_scatter(zb, [j * 2 + r_off, c_off], zeros)
        return carry
    lax.fori_loop(0, 32, zfill, 0)
    for t in range(9):
        pltpu.sync_copy(zb, s_sh.at[pl.ds(r0 + t * 64, 64)])
    pltpu.sync_copy(zb, s_sh.at[pl.ds(r_tail, 64)])

    cb = wid * NCHUNK
    pltpu.sync_copy(row_hbm.at[pl.ds(cb, NCHUNK)], rows_all)
    pltpu.sync_copy(col_hbm.at[pl.ds(cb, NCHUNK)], cols_all)
    pltpu.sync_copy(a_hbm.at[pl.ds(cb, NCHUNK)], a_all)
    plsc.subcore_barrier()

    ebase = wid * (NCHUNK * C)

    def issue_in(b, c):
        pltpu.async_copy(q_hbm.at[rows_all.at[c]], qrows[b], semi[b])
        pltpu.async_copy(k_hbm.at[cols_all.at[c]], krows[b], semi[b])
        pltpu.async_copy(m_hbm.at[rows_all.at[c]], mrows[b], semi[b])

    def wait_in(b, c):
        pltpu.make_async_copy(q_hbm.at[rows_all.at[c]], qrows[b], semi[b]).wait()
        pltpu.make_async_copy(k_hbm.at[cols_all.at[c]], krows[b], semi[b]).wait()
        pltpu.make_async_copy(m_hbm.at[rows_all.at[c]], mrows[b], semi[b]).wait()

    def do_out(b, c):
        base = pl.multiple_of(ebase + c * C, 8)
        pltpu.sync_copy(exb[b], ex_hbm.at[pl.ds(base, C)])
        pltpu.sync_copy(exb[b], s_sh.at[rows_all.at[c]], add=True)

    def compute(b, c):
        def blk_body(j, bcarry):
            e_idx = j * LANES + iot
            a_vr = a_all[c, pl.ds(j * LANES, LANES)]
            for hd in range(HEADS):
                acc = jnp.zeros((LANES,), jnp.float32)
                for d in range(HEAD_DIM):
                    ch = jnp.full((LANES,), hd * HEAD_DIM + d, jnp.int32)
                    qT = plsc.load_gather(qrows[b], [e_idx, ch])
                    kT = plsc.load_gather(krows[b], [e_idx, ch])
                    acc = acc + qT * kT
                hvec = jnp.full((LANES,), hd, jnp.int32)
                mvr = plsc.load_gather(mrows[b], [e_idx, hvec])
                ex = jnp.exp(acc * a_vr - mvr)
                plsc.store_scatter(exb[b], [e_idx, hvec], ex)
            return bcarry
        lax.fori_loop(0, C // LANES, blk_body, 0)

    issue_in(0, 0)
    issue_in(1, 1)

    def g_body(g, carry):
        for b in range(2):
            c = g * 2 + b
            wait_in(b, c)
            compute(b, c)
            do_out(b, c)
            if b == 0:
                issue_in(b, c + 2)
            else:
                @pl.when(g <= (NCHUNK - 1) // 2 - 2)
                def _():
                    issue_in(b, c + 2)
        return carry
    lax.fori_loop(0, (NCHUNK - 1) // 2, g_body, 0)

    wait_in(0, NCHUNK - 1)
    compute(0, NCHUNK - 1)
    do_out(0, NCHUNK - 1)

    plsc.subcore_barrier()
    obase = pl.multiple_of(cid * n + r0, 8)
    otail = pl.multiple_of(cid * n + r_tail, 8)
    for t in range(9):
        pltpu.sync_copy(s_sh.at[pl.ds(r0 + t * 64, 64)],
                        spart_hbm.at[pl.ds(obase + t * 64, 64)])
    pltpu.sync_copy(s_sh.at[pl.ds(r_tail, 64)],
                    spart_hbm.at[pl.ds(otail, 64)])


def _sc_a(row2, col2, a2, q, k, m):
    n = q.shape[0]
    n_edges = row2.shape[0] * row2.shape[1]
    f32 = jnp.float32
    i32 = jnp.int32
    mesh = plsc.VectorSubcoreMesh(core_axis_name="c", subcore_axis_name="s")
    return pl.kernel(
        _sc_a_body,
        compiler_params=_SC_PARAMS,
        out_type=(
            jax.ShapeDtypeStruct((n_edges, HEADS + 1), f32),
            jax.ShapeDtypeStruct((2 * n, HEADS + 1), f32),
        ),
        mesh=mesh,
        scratch_types=[
            pltpu.VMEM((NCHUNK, C), i32),
            pltpu.VMEM((NCHUNK, C), i32),
            pltpu.VMEM((NCHUNK, C), f32),
            pltpu.VMEM((C, HIDDEN + 1), f32),
            pltpu.VMEM((C, HIDDEN + 1), f32),
            pltpu.VMEM((C, HIDDEN + 1), f32),
            pltpu.VMEM((C, HIDDEN + 1), f32),
            pltpu.VMEM((C, HEADS + 1), f32),
            pltpu.VMEM((C, HEADS + 1), f32),
            pltpu.VMEM((C, HEADS + 1), f32),
            pltpu.VMEM((C, HEADS + 1), f32),
            pltpu.VMEM((64, HEADS + 1), f32),
            pltpu.VMEM_SHARED((n, HEADS + 1), f32),
            pltpu.SemaphoreType.DMA,
            pltpu.SemaphoreType.DMA,
        ],
    )(row2, col2, a2, q, k, m)


def _sc_b_body(row_hbm, col_hbm, ex_hbm, s0_hbm, s1_hbm, v_hbm,
               opart_hbm,
               rb0, rb1, rb2, rb3, cb0, cb1, cb2, cb3,
               vrows0, vrows1, s0r0, s0r1, s1r0, s1r1, exb0, exb1,
               con0, con1, zb, out_sh,
               semx0, semx1, semx2, semx3, semi0, semi1):
    n = v_hbm.shape[0]
    rowb = (rb0, rb1, rb2, rb3)
    colb = (cb0, cb1, cb2, cb3)
    vrows = (vrows0, vrows1)
    s0r = (s0r0, s0r1)
    s1r = (s1r0, s1r1)
    exb = (exb0, exb1)
    con = (con0, con1)
    semx = (semx0, semx1, semx2, semx3)
    semi = (semi0, semi1)

    cid = lax.axis_index("c")
    sid = lax.axis_index("s")
    wid = sid * NC + cid

    iot = lax.iota(jnp.int32, LANES)
    r_off = iot // HEADS
    c_off = iot % HEADS
    zeros = jnp.zeros((LANES,), jnp.float32)

    tiles = n // 8
    tpw = tiles // NS
    extra = tiles - tpw * NS
    t0 = sid * tpw + jnp.minimum(sid, extra)
    myt = tpw + jnp.where(sid < extra, 1, 0)
    r0 = pl.multiple_of(t0 * 8, 8)
    r_tail = pl.multiple_of((t0 + myt) * 8 - 8, 8)

    def zfill(r, carry):
        for t in range(HEADS):
            zb[r, pl.ds(t * LANES, LANES)] = zeros
        return carry
    lax.fori_loop(0, 8, zfill, 0)
    for t in range(78):
        pltpu.sync_copy(zb, out_sh.at[pl.ds(r0 + t * 8, 8)])
    pltpu.sync_copy(zb, out_sh.at[pl.ds(r_tail, 8)])
    plsc.subcore_barrier()

    cbase = wid * NCHUNK
    ebase = wid * (NCHUNK * C)

    def issue_idx(s, c):
        pltpu.async_copy(row_hbm.at[cbase + c], rowb[s], semx[s])
        pltpu.async_copy(col_hbm.at[cbase + c], colb[s], semx[s])

    def wait_idx(s, c):
        pltpu.make_async_copy(row_hbm.at[cbase + c], rowb[s], semx[s]).wait()
        pltpu.make_async_copy(col_hbm.at[cbase + c], colb[s], semx[s]).wait()

    def issue_in(b, s, c):
        base = pl.multiple_of(ebase + c * C, 8)
        pltpu.async_copy(v_hbm.at[colb[s]], vrows[b], semi[b])
        pltpu.async_copy(s0_hbm.at[rowb[s]], s0r[b], semi[b])
        pltpu.async_copy(s1_hbm.at[rowb[s]], s1r[b], semi[b])
        pltpu.async_copy(ex_hbm.at[pl.ds(base, C)], exb[b], semi[b])

    def wait_in(b, s, c):
        base = pl.multiple_of(ebase + c * C, 8)
        pltpu.make_async_copy(v_hbm.at[colb[s]], vrows[b], semi[b]).wait()
        pltpu.make_async_copy(s0_hbm.at[rowb[s]], s0r[b], semi[b]).wait()
        pltpu.make_async_copy(s1_hbm.at[rowb[s]], s1r[b], semi[b]).wait()
        pltpu.make_async_copy(ex_hbm.at[pl.ds(base, C)], exb[b], semi[b]).wait()

    def do_out(b, s):
        pltpu.sync_copy(con[b], out_sh.at[rowb[s]], add=True)

    def compute(b):
        def vbody(j, vcarry):
            idx_r = j * 2 + r_off
            ex = plsc.load_gather(exb[b], [idx_r, c_off])
            s0 = plsc.load_gather(s0r[b], [idx_r, c_off])
            s1 = plsc.load_gather(s1r[b], [idx_r, c_off])
            at = ex / jnp.maximum(s0 + s1, 1e-9)
            plsc.store_scatter(exb[b], [idx_r, c_off], at)
            return vcarry
        lax.fori_loop(0, C * HEADS // LANES, vbody, 0)

        def pair_body(j, pcarry):
            at16 = plsc.load_gather(exb[b], [j * 2 + r_off, c_off])
            for par in range(2):
                e = j * 2 + par
                for hd in range(HEADS):
                    a = at16[par * HEADS + hd]
                    con[b][e, pl.ds(hd * HEAD_DIM, HEAD_DIM)] = (
                        vrows[b][e, pl.ds(hd * HEAD_DIM, HEAD_DIM)] * a)
            return pcarry
        lax.fori_loop(0, C // 2, pair_body, 0)

    issue_idx(0, 0)
    issue_idx(1, 1)
    wait_idx(0, 0)
    issue_in(0, 0, 0)

    def g_body(g, carry):
        for u in range(4):
            c = g * 4 + u
            b = u % 2
            @pl.when(c <= NCHUNK - 3)
            def _():
                issue_idx((u + 2) % 4, c + 2)
            wait_in(b, u, c)
            compute(b)
            do_out(b, u)
            wait_idx((u + 1) % 4, c + 1)
            issue_in(1 - b, (u + 1) % 4, c + 1)
        return carry
    lax.fori_loop(0, (NCHUNK - 1) // 4, g_body, 0)

    wait_in(0, 0, NCHUNK - 1)
    compute(0)
    do_out(0, 0)

    plsc.subcore_barrier()
    obase = pl.multiple_of(cid * n + r0, 8)
    otail = pl.multiple_of(cid * n + r_tail, 8)
    for t in range(19):
        pltpu.sync_copy(out_sh.at[pl.ds(r0 + t * 32, 32)],
                        opart_hbm.at[pl.ds(obase + t * 32, 32)])
    pltpu.sync_copy(out_sh.at[pl.ds(r_tail, 32)],
                    opart_hbm.at[pl.ds(otail, 32)])


def _sc_b(row2, col2, ex, s0, s1, v):
    n = v.shape[0]
    f32 = jnp.float32
    i32 = jnp.int32
    mesh = plsc.VectorSubcoreMesh(core_axis_name="c", subcore_axis_name="s")
    return pl.kernel(
        _sc_b_body,
        compiler_params=_SC_PARAMS,
        out_type=jax.ShapeDtypeStruct((2 * n, HIDDEN), f32),
        mesh=mesh,
        scratch_types=(
            [pltpu.VMEM((C,), i32) for _ in range(8)]
            + [
                pltpu.VMEM((C, HIDDEN), f32),
                pltpu.VMEM((C, HIDDEN), f32),
                pltpu.VMEM((C, HEADS + 1), f32),
                pltpu.VMEM((C, HEADS + 1), f32),
                pltpu.VMEM((C, HEADS + 1), f32),
                pltpu.VMEM((C, HEADS + 1), f32),
                pltpu.VMEM((C, HEADS + 1), f32),
                pltpu.VMEM((C, HEADS + 1), f32),
                pltpu.VMEM((C, HIDDEN), f32),
                pltpu.VMEM((C, HIDDEN), f32),
                pltpu.VMEM((8, HIDDEN), f32),
                pltpu.VMEM_SHARED((n, HIDDEN), f32),
            ]
            + [pltpu.SemaphoreType.DMA for _ in range(6)]
        ),
    )(row2, col2, ex, s0, s1, v)


def kernel(h, edge_index, A_val, Wq, bq, Wk, bk, Wv, bv, Wo, bo):
    n = h.shape[0]
    perm = jnp.asarray(_PERM)
    wq = (Wq * SCALING)[perm, :]
    bqp = (bq * SCALING)[perm]
    wk = Wk[perm, :]
    bkp = bk[perm]
    wv = Wv[perm, :]
    bvp = bv[perm]
    w_cat = jnp.concatenate([wq.T, wk.T, wv.T], axis=1)
    b_cat = jnp.concatenate([bqp, bkp, bvp])[None, :]
    sel = jnp.asarray(
        (np.arange(HIDDEN)[:, None] // HEAD_DIM == np.arange(HEADS)[None, :])
        .astype(np.float32))
    a2d = A_val.reshape(-1, HIDDEN)

    q, k, v, m = _dense1(h, w_cat, b_cat, sel, a2d)

    row2 = edge_index[0].reshape(NW * NCHUNK, C)
    col2 = edge_index[1].reshape(NW * NCHUNK, C)
    a2 = A_val.reshape(NW * NCHUNK, C)
    ex, spart = _sc_a(row2, col2, a2, q, k, m)
    opart = _sc_b(row2, col2, ex, spart[:n], spart[n:], v)

    w_eff = Wo.T[perm, :]
    return _dense2(opart[:n], opart[n:], w_eff, bo[None, :])
